# baseline (speedup 1.0000x reference)
import functools

import jax
import jax.numpy as jnp
from jax import lax
from jax.experimental import pallas as pl
from jax.experimental.pallas import tpu as pltpu
from jax.experimental.pallas import tpu_sc as plsc

NC = 2
NS = 16
NW = NC * NS
BATCH = 16384
D = 16
B_PER_W = BATCH // NW
CHUNK = 128
NCHUNK = B_PER_W // CHUNK


def _sc_body(uid_idx_hbm, iid_idx_hbm, uid_hbm, iid_hbm, out_hbm,
             idxu_v, idxi_v, rows_u, rows_i, out_v, sem_u, sem_i):
    wid = lax.axis_index("s") * NC + lax.axis_index("c")
    base = wid * B_PER_W

    pltpu.sync_copy(uid_idx_hbm.at[wid], idxu_v)
    pltpu.sync_copy(iid_idx_hbm.at[wid], idxi_v)

    copies = []
    for j in range(NCHUNK):
        copies.append(pltpu.async_copy(
            uid_hbm.at[idxu_v.at[j]], rows_u.at[pl.ds(j * CHUNK, CHUNK)],
            sem_u))
        copies.append(pltpu.async_copy(
            iid_hbm.at[idxi_v.at[j]], rows_i.at[pl.ds(j * CHUNK, CHUNK)],
            sem_i))
    for c in copies:
        c.wait()

    lanes = lax.iota(jnp.int32, 16)

    def group(g, _):
        acc = jnp.zeros((16,), jnp.float32)
        for k in range(16):
            r = g * 16 + k
            p = rows_u[r, :] * rows_i[r, :]
            s = jnp.sum(p)
            acc = jnp.where(lanes == k, jnp.broadcast_to(s, (16,)), acc)
        out_v[pl.ds(g * 16, 16)] = acc
        return _

    lax.fori_loop(0, B_PER_W // 16, group, None)

    pltpu.sync_copy(out_v, out_hbm.at[pl.ds(base, B_PER_W)])


@jax.jit
def _mf_score(uid_idx, iid_idx, uid_table, iid_table):
    mesh = plsc.VectorSubcoreMesh(core_axis_name="c", subcore_axis_name="s")
    fn = functools.partial(
        pl.kernel,
        mesh=mesh,
        out_type=jax.ShapeDtypeStruct((BATCH,), jnp.float32),
        compiler_params=pltpu.CompilerParams(
            needs_layout_passes=False, use_tc_tiling_on_sc=False),
        scratch_types=[
            pltpu.VMEM((NCHUNK, CHUNK), jnp.int32),
            pltpu.VMEM((NCHUNK, CHUNK), jnp.int32),
            pltpu.VMEM((B_PER_W, D), jnp.float32),
            pltpu.VMEM((B_PER_W, D), jnp.float32),
            pltpu.VMEM((B_PER_W,), jnp.float32),
            pltpu.SemaphoreType.DMA,
            pltpu.SemaphoreType.DMA,
        ],
    )(_sc_body)
    return fn(uid_idx, iid_idx, uid_table, iid_table)


def kernel(x, uid_table, iid_table):
    uid_idx = x[:, 0].reshape(NW, NCHUNK, CHUNK)
    iid_idx = x[:, 1].reshape(NW, NCHUNK, CHUNK)
    return _mf_score(uid_idx, iid_idx, uid_table, iid_table)


# speedup vs baseline: 5.9246x; 5.9246x over previous
import functools

import jax
import jax.numpy as jnp
from jax import lax
from jax.experimental import pallas as pl
from jax.experimental.pallas import tpu as pltpu
from jax.experimental.pallas import tpu_sc as plsc

NC = 2
NS = 16
NW = NC * NS
BATCH = 16384
D = 16
W = 128
B_PER_W = BATCH // NW
G = 16
NGROUP = B_PER_W // G


def _sc_body(uidx_hbm, iidx_hbm, ut_hbm, it_hbm, out_hbm,
             col_v, wof_v, win, ucols, icols, out_v, sem0, sem1):
    wid = lax.axis_index("s") * NC + lax.axis_index("c")
    base = wid * B_PER_W

    pltpu.sync_copy(uidx_hbm.at[wid], col_v.at[0])
    pltpu.sync_copy(iidx_hbm.at[wid], col_v.at[1])
    for j in range(B_PER_W // 16):
        s = pl.ds(j * 16, 16)
        for row in (0, 1):
            raw = col_v[row, s]
            wof_v[row, s] = jnp.right_shift(raw, 7) * W
            col_v[row, s] = jnp.bitwise_and(raw, W - 1)

    kiota = lax.iota(jnp.int32, 16)
    sems = (sem0, sem1)

    def phase(tab_hbm, col_row, cols):
        def enqueue(g, buf):
            wvec = wof_v[col_row, pl.ds(g * G, G)]
            for k in range(G):
                c = pl.multiple_of(wvec[k], W)
                pltpu.async_copy(
                    tab_hbm.at[:, pl.ds(c, W)], win.at[buf, k],
                    sems[buf])

        def handle(g, buf):
            for k in range(G):
                pltpu.make_async_copy(
                    tab_hbm.at[:, pl.ds(0, W)], win.at[buf, k],
                    sems[buf]).wait()
            colv = col_v[col_row, pl.ds(g * G, G)]
            for d in range(D):
                dvec = jnp.broadcast_to(jnp.int32(d), (16,))
                cols[d, pl.ds(g * G, G)] = plsc.load_gather(
                    win.at[buf], [kiota, dvec, colv])

            @pl.when(g + 2 < NGROUP)
            def _():
                enqueue(g + 2, buf)

        enqueue(0, 0)
        enqueue(1, 1)

        def body(i, _):
            handle(2 * i, 0)
            handle(2 * i + 1, 1)
            return _

        lax.fori_loop(0, NGROUP // 2, body, None)

    phase(ut_hbm, 0, ucols)
    phase(it_hbm, 1, icols)

    def dot(j, _):
        s = pl.ds(j * G, G)
        acc = ucols[0, s] * icols[0, s]
        for d in range(1, D):
            acc = acc + ucols[d, s] * icols[d, s]
        out_v[s] = acc
        return _

    lax.fori_loop(0, NGROUP, dot, None)

    pltpu.sync_copy(out_v, out_hbm.at[pl.ds(base, B_PER_W)])


@jax.jit
def _mf_score(uid_idx, iid_idx, ut, it):
    mesh = plsc.VectorSubcoreMesh(core_axis_name="c", subcore_axis_name="s")
    fn = functools.partial(
        pl.kernel,
        mesh=mesh,
        out_type=jax.ShapeDtypeStruct((BATCH,), jnp.float32),
        compiler_params=pltpu.CompilerParams(needs_layout_passes=False),
        scratch_types=[
            pltpu.VMEM((2, B_PER_W), jnp.int32),
            pltpu.VMEM((2, B_PER_W), jnp.int32),
            pltpu.VMEM((2, G, D, W), jnp.float32),
            pltpu.VMEM((D, B_PER_W), jnp.float32),
            pltpu.VMEM((D, B_PER_W), jnp.float32),
            pltpu.VMEM((B_PER_W,), jnp.float32),
            pltpu.SemaphoreType.DMA,
            pltpu.SemaphoreType.DMA,
        ],
    )(_sc_body)
    return fn(uid_idx, iid_idx, ut, it)


def kernel(x, uid_table, iid_table):
    uid_idx = x[:, 0].reshape(NW, B_PER_W)
    iid_idx = x[:, 1].reshape(NW, B_PER_W)
    return _mf_score(uid_idx, iid_idx, uid_table.T, iid_table.T)
